# baseline (speedup 1.0000x reference)
import jax
import jax.numpy as jnp
from jax.experimental import pallas as pl

_TEMPERATURE = 64.0
_GRAM_BLOCK = 120


def _protonet_body(s_ref, q_ref, o_ref):
    inv_t = 1.0 / _TEMPERATURE
    proto = jnp.sum(s_ref[...], axis=0) * (1.0 / s_ref.shape[0])
    pn = (jnp.sum(proto * proto, axis=1) * inv_t)[None, :]
    w = proto * (2.0 * inv_t)

    nb = _GRAM_BLOCK
    row = jax.lax.broadcasted_iota(jnp.int32, (nb, nb), 0)
    diag_mask = (row == jax.lax.broadcasted_iota(jnp.int32, (nb, nb), 1))

    for b in range(q_ref.shape[0] // nb):
        qb = q_ref[pl.ds(b * nb, nb), :]
        cross = jax.lax.dot_general(
            qb, w, (((1,), (1,)), ((), ())),
            preferred_element_type=jnp.float32,
        )
        gram = jax.lax.dot_general(
            qb, qb, (((1,), (1,)), ((), ())),
            preferred_element_type=jnp.float32,
        )
        qn = jnp.sum(jnp.where(diag_mask, gram, 0.0), axis=1,
                     keepdims=True) * inv_t
        o_ref[pl.ds(b * nb, nb), :] = cross - qn - pn


def kernel(support, query):
    n_batch, n_shot, n_way, emb_dim = support.shape
    n_query = n_batch * query.shape[1] * n_way
    s = support.reshape(n_shot, n_way, emb_dim)
    q = query.reshape(n_query, emb_dim)
    return pl.pallas_call(
        _protonet_body,
        out_shape=jax.ShapeDtypeStruct((n_query, n_way), jnp.float32),
    )(s, q)


# speedup vs baseline: 1.1342x; 1.1342x over previous
import jax
import jax.numpy as jnp
from jax.experimental import pallas as pl
from jax.experimental.pallas import tpu as pltpu

_TEMPERATURE = 64.0
_N_SPLIT = 2


def _protonet_body(s_ref, q_ref, o_ref):
    inv_t = 1.0 / _TEMPERATURE
    proto = jnp.sum(s_ref[...], axis=0) * (1.0 / s_ref.shape[0])
    q = q_ref[...]
    qn = jnp.sum(q * q, axis=1, keepdims=True) * inv_t
    pn = (jnp.sum(proto * proto, axis=1) * inv_t)[None, :]
    cross = jax.lax.dot_general(
        q, proto * (2.0 * inv_t), (((1,), (1,)), ((), ())),
        preferred_element_type=jnp.float32,
    )
    o_ref[...] = cross - qn - pn


def kernel(support, query):
    n_batch, n_shot, n_way, emb_dim = support.shape
    n_query = n_batch * query.shape[1] * n_way
    blk = n_query // _N_SPLIT
    s = support.reshape(n_shot, n_way, emb_dim)
    q = query.reshape(n_query, emb_dim)
    return pl.pallas_call(
        _protonet_body,
        grid=(_N_SPLIT,),
        in_specs=[
            pl.BlockSpec((n_shot, n_way, emb_dim), lambda i: (0, 0, 0)),
            pl.BlockSpec((blk, emb_dim), lambda i: (i, 0)),
        ],
        out_specs=pl.BlockSpec((blk, n_way), lambda i: (i, 0)),
        out_shape=jax.ShapeDtypeStruct((n_query, n_way), jnp.float32),
        compiler_params=pltpu.CompilerParams(
            dimension_semantics=("parallel",),
        ),
    )(s, q)
